# baseline (speedup 1.0000x reference)
import jax
import jax.numpy as jnp
from jax import lax
from jax.experimental import pallas as pl
from jax.experimental.pallas import tpu as pltpu
from jax.experimental.pallas import tpu_sc as plsc

B = 16384
D = 16
V = 100000

_INFO = plsc.get_sparse_core_info()
_NC = _INFO.num_cores
_NS = _INFO.num_subcores
_NW = _NC * _NS
_BPW = B // _NW
_Q = 128
_NQ = _BPW // _Q
_NSTEP = 2 * _NQ
_NBUF = 6


def _body(contrib_table, recip_table, contrib_idx, recip_idx,
          xr_out, xc_out,
          idx_rv, idx_cv,
          b0, b1, b2, b3, b4, b5,
          sem_ir, sem_ic, sem_gr, sem_gc, sem_wr, sem_wc):
    wid = lax.axis_index("s") * _NC + lax.axis_index("c")
    base = wid * _BPW
    sl = pl.ds(base, _BPW)
    ir = pltpu.async_copy(recip_idx.at[sl], idx_rv, sem_ir)
    ic = pltpu.async_copy(contrib_idx.at[sl], idx_cv, sem_ic)
    ir.wait()
    ic.wait()

    bufs = (b0, b1, b2, b3, b4, b5)
    tabs = (recip_table, contrib_table)
    idxs = (idx_rv, idx_cv)
    outs = (xr_out, xc_out)
    gsems = (sem_gr, sem_gc)
    wsems = (sem_wr, sem_wc)

    def issue(k):
        t, q, buf = k % 2, k // 2, bufs[k % _NBUF]
        tab, idx_v = tabs[t], idxs[t]

        def grp(g, _):
            j0 = q * _Q + g * 16
            v = idx_v[pl.ds(j0, 16)]
            for l in range(16):
                pltpu.async_copy(tab.at[pl.ds(v[l], 1)],
                                 buf.at[pl.ds(g * 16 + l, 1)], gsems[t])
            return 0

        lax.fori_loop(0, _Q // 16, grp, 0)

    def drain_write(k):
        t, q, buf = k % 2, k // 2, bufs[k % _NBUF]
        o = pl.ds(base + q * _Q, _Q)
        pltpu.make_async_copy(outs[t].at[o], buf, gsems[t]).wait()
        return pltpu.async_copy(buf, outs[t].at[o], wsems[t])

    wh = [None] * _NSTEP
    for k in range(_NSTEP):
        if k >= _NBUF:
            d = k - _NBUF
            wh[d] = drain_write(d)
            wh[d].wait()
        issue(k)
    for d in range(_NSTEP):
        if wh[d] is None:
            wh[d] = drain_write(d)
    for d in range(_NSTEP - _NBUF, _NSTEP):
        wh[d].wait()


@jax.jit
def kernel(contrib_table, recip_table, contrib_idx, recip_idx):
    mesh = plsc.VectorSubcoreMesh(core_axis_name="c", subcore_axis_name="s")
    xr, xc = pl.kernel(
        _body,
        mesh=mesh,
        out_type=(
            jax.ShapeDtypeStruct((B, D), jnp.float32),
            jax.ShapeDtypeStruct((B, D), jnp.float32),
        ),
        scratch_types=[
            pltpu.VMEM((_BPW,), jnp.int32),
            pltpu.VMEM((_BPW,), jnp.int32),
            pltpu.VMEM((_Q, D), jnp.float32),
            pltpu.VMEM((_Q, D), jnp.float32),
            pltpu.VMEM((_Q, D), jnp.float32),
            pltpu.VMEM((_Q, D), jnp.float32),
            pltpu.VMEM((_Q, D), jnp.float32),
            pltpu.VMEM((_Q, D), jnp.float32),
            pltpu.SemaphoreType.DMA,
            pltpu.SemaphoreType.DMA,
            pltpu.SemaphoreType.DMA,
            pltpu.SemaphoreType.DMA,
            pltpu.SemaphoreType.DMA,
            pltpu.SemaphoreType.DMA,
        ],
        compiler_params=pltpu.CompilerParams(needs_layout_passes=False),
    )(contrib_table, recip_table, contrib_idx, recip_idx)
    return xr, xc
